# baseline (speedup 1.0000x reference)
import functools

import jax
import jax.numpy as jnp
from jax import lax
from jax.experimental import pallas as pl
from jax.experimental.pallas import tpu as pltpu
from jax.experimental.pallas import tpu_sc as plsc

_NUM_FIELDS = 26
_PER_FIELD_VOCAB = 100000
_EMBED_DIM = 32
_BATCH = 16384

_NC = 2
_NS = 16
_L = 16
_NW = _NC * _NS
_TOTAL = _BATCH * _NUM_FIELDS
_B_PER_W = _TOTAL // _NW
_CHUNK = 1664
_NCHUNK = _B_PER_W // _CHUNK
_GROUPS = _CHUNK // _L
_FIELD_WRAP = _NUM_FIELDS * _PER_FIELD_VOCAB
_STEP = _L * _PER_FIELD_VOCAB

_mesh = plsc.VectorSubcoreMesh(core_axis_name="c", subcore_axis_name="s")


@functools.partial(
    pl.kernel,
    out_type=jax.ShapeDtypeStruct((_TOTAL, _EMBED_DIM), jnp.float32),
    mesh=_mesh,
    compiler_params=pltpu.CompilerParams(use_tc_tiling_on_sc=False),
    scratch_types=[
        pltpu.VMEM((_CHUNK,), jnp.int32),
        pltpu.VMEM((_CHUNK,), jnp.int32),
        pltpu.VMEM((_CHUNK, _EMBED_DIM), jnp.float32),
        pltpu.SemaphoreType.DMA,
    ],
)
def _emb_lookup(x_hbm, table_hbm, out_hbm, raw_v, idx_v, rows_v, sem):
    wid = lax.axis_index("s") * _NC + lax.axis_index("c")
    base = wid * _B_PER_W

    def chunk_body(c, carry):
        cbase = base + c * _CHUNK
        pltpu.sync_copy(x_hbm.at[pl.ds(cbase, _CHUNK)], raw_v)

        def group_body(g, offs):
            sl = pl.ds(g * _L, _L)
            idx_v[sl] = raw_v[sl] + offs
            nxt = offs + _STEP
            return jnp.where(nxt >= _FIELD_WRAP, nxt - _FIELD_WRAP, nxt)

        offs0 = lax.iota(jnp.int32, _L) * _PER_FIELD_VOCAB
        lax.fori_loop(0, _GROUPS, group_body, offs0)

        pltpu.async_copy(table_hbm.at[idx_v], rows_v, sem).wait()
        pltpu.sync_copy(rows_v, out_hbm.at[pl.ds(cbase, _CHUNK)])
        return carry

    lax.fori_loop(0, _NCHUNK, chunk_body, 0)


@jax.jit
def kernel(x, embedding_table):
    out = _emb_lookup(x.reshape(_TOTAL), embedding_table)
    return out.reshape(_BATCH, _NUM_FIELDS, _EMBED_DIM)
